# Initial kernel scaffold; baseline (speedup 1.0000x reference)
import jax
import jax.numpy as jnp
from jax.experimental import pallas as pl


def kernel(indices, glove_table, weight):
    raise NotImplementedError("write your pallas kernel here")


# speedup vs baseline: 1.5167x; 1.5167x over previous
import jax
import jax.numpy as jnp
from jax import lax
from jax.experimental import pallas as pl
from jax.experimental.pallas import tpu as pltpu
from jax.experimental.pallas import tpu_sc as plsc

V = 1000000
D = 100
DP = 112
B = 16384
L = 200

NC = 2
NS = 16
NW = NC * NS

N = B * L
PER_W = N // NW
CHUNK = 512
SUB = 128
NSUB = CHUNK // SUB
STEPS = PER_W // CHUNK
IDX_ROWS = N // SUB

ADD_BLOCK = 8192
ADD_GRID = -(-V // ADD_BLOCK)


def _add_body(a_ref, b_ref, o_ref):
    o_ref[:, :D] = a_ref[...] + b_ref[...]
    o_ref[:, D:] = jnp.zeros((ADD_BLOCK, DP - D), jnp.float32)


def _combine(glove_table, weight):
    return pl.pallas_call(
        _add_body,
        out_shape=jax.ShapeDtypeStruct((V, DP), jnp.float32),
        grid=(ADD_GRID,),
        in_specs=[
            pl.BlockSpec((ADD_BLOCK, D), lambda i: (i, 0)),
            pl.BlockSpec((ADD_BLOCK, D), lambda i: (i, 0)),
        ],
        out_specs=pl.BlockSpec((ADD_BLOCK, DP), lambda i: (i, 0)),
    )(glove_table, weight)


def _gather_body(idx_hbm, tbl_hbm, out_hbm, idx_v, rows_v, sem):
    wid = lax.axis_index("s") * NC + lax.axis_index("c")

    def step(g, carry):
        idx_row = wid * (PER_W // SUB) + g * NSUB
        out_row = wid * PER_W + g * CHUNK
        pltpu.sync_copy(idx_hbm.at[pl.ds(idx_row, NSUB)], idx_v)
        cps = [
            pltpu.async_copy(
                tbl_hbm.at[idx_v.at[j]],
                rows_v.at[pl.ds(j * SUB, SUB)],
                sem,
            )
            for j in range(NSUB)
        ]
        for c in cps:
            c.wait()
        pltpu.sync_copy(rows_v, out_hbm.at[pl.ds(out_row, CHUNK)])
        return carry

    lax.fori_loop(0, STEPS, step, 0)


def kernel(indices, glove_table, weight):
    combined = _combine(glove_table, weight)
    idx2d = indices.reshape(IDX_ROWS, SUB).astype(jnp.int32)
    out = pl.kernel(
        _gather_body,
        out_type=jax.ShapeDtypeStruct((N, DP), jnp.float32),
        mesh=plsc.VectorSubcoreMesh(core_axis_name="c", subcore_axis_name="s"),
        compiler_params=pltpu.CompilerParams(use_tc_tiling_on_sc=False),
        scratch_types=[
            pltpu.VMEM((NSUB, SUB), jnp.int32),
            pltpu.VMEM((CHUNK, DP), jnp.float32),
            pltpu.SemaphoreType.DMA,
        ],
    )(idx2d, combined)
    return out[:, :D].reshape(B, L, D)


# speedup vs baseline: 2.4104x; 1.5893x over previous
import jax
import jax.numpy as jnp
from jax import lax
from jax.experimental import pallas as pl
from jax.experimental.pallas import tpu as pltpu
from jax.experimental.pallas import tpu_sc as plsc

V = 1000000
D = 100
DP = 128
B = 16384
L = 200

NC = 2
NS = 16
NW = NC * NS

N = B * L
PER_W = N // NW
CHUNK = 512
SUB = 128
NSUB = CHUNK // SUB
STEPS = PER_W // CHUNK
IDX_T = N // (8 * SUB)

ADD_BLOCK = 8192
ADD_GRID = -(-V // ADD_BLOCK)


def _add_body(a_ref, b_ref, o_ref):
    o_ref[:, :D] = a_ref[...] + b_ref[...]


def _combine(glove_table, weight):
    return pl.pallas_call(
        _add_body,
        out_shape=jax.ShapeDtypeStruct((V, DP), jnp.float32),
        grid=(ADD_GRID,),
        in_specs=[
            pl.BlockSpec((ADD_BLOCK, D), lambda i: (i, 0)),
            pl.BlockSpec((ADD_BLOCK, D), lambda i: (i, 0)),
        ],
        out_specs=pl.BlockSpec((ADD_BLOCK, DP), lambda i: (i, 0)),
    )(glove_table, weight)


def _gather_body(idx_hbm, tbl_hbm, out_hbm, idx_v, rows_v, sem):
    wid = lax.axis_index("s") * NC + lax.axis_index("c")

    def step(g, carry):
        out_row = wid * PER_W + g * CHUNK

        @pl.when(g % 2 == 0)
        def _():
            tile = wid * (STEPS // 2) + g // 2
            pltpu.sync_copy(idx_hbm.at[tile], idx_v)

        half = (g % 2) * NSUB
        cps = [
            pltpu.async_copy(
                tbl_hbm.at[idx_v.at[half + j]],
                rows_v.at[pl.ds(j * SUB, SUB)],
                sem,
            )
            for j in range(NSUB)
        ]
        for c in cps:
            c.wait()
        pltpu.sync_copy(rows_v, out_hbm.at[pl.ds(out_row, CHUNK)])
        return carry

    lax.fori_loop(0, STEPS, step, 0)


def kernel(indices, glove_table, weight):
    combined = _combine(glove_table, weight)
    idx3d = indices.reshape(IDX_T, 8, SUB).astype(jnp.int32)
    out = pl.kernel(
        _gather_body,
        out_type=jax.ShapeDtypeStruct((N, DP), jnp.float32),
        mesh=plsc.VectorSubcoreMesh(core_axis_name="c", subcore_axis_name="s"),
        compiler_params=pltpu.CompilerParams(use_tc_tiling_on_sc=True),
        scratch_types=[
            pltpu.VMEM((8, SUB), jnp.int32),
            pltpu.VMEM((CHUNK, DP), jnp.float32),
            pltpu.SemaphoreType.DMA,
        ],
    )(idx3d, combined)
    return out[:, :D].reshape(B, L, D)


# speedup vs baseline: 2.5290x; 1.0492x over previous
import jax
import jax.numpy as jnp
from jax import lax
from jax.experimental import pallas as pl
from jax.experimental.pallas import tpu as pltpu
from jax.experimental.pallas import tpu_sc as plsc

V = 1000000
D = 100
DP = 128
B = 16384
L = 200

NC = 2
NS = 16
NW = NC * NS

N = B * L
PER_W = N // NW
CHUNK = 128
SUB = 128
NBUF = 4
STEPS = PER_W // CHUNK
UNROLL = 16
IDX_T = N // (8 * SUB)

ADD_BLOCK = 8192
ADD_GRID = -(-V // ADD_BLOCK)


def _add_body(a_ref, b_ref, o_ref):
    o_ref[:, :D] = a_ref[...] + b_ref[...]


def _combine(glove_table, weight):
    return pl.pallas_call(
        _add_body,
        out_shape=jax.ShapeDtypeStruct((V, DP), jnp.float32),
        grid=(ADD_GRID,),
        in_specs=[
            pl.BlockSpec((ADD_BLOCK, D), lambda i: (i, 0)),
            pl.BlockSpec((ADD_BLOCK, D), lambda i: (i, 0)),
        ],
        out_specs=pl.BlockSpec((ADD_BLOCK, DP), lambda i: (i, 0)),
    )(glove_table, weight)


def _gather_body(idx_hbm, tbl_hbm, out_hbm, idx_v, rows_v,
                 sg0, sg1, sg2, sg3, so0, so1, so2, so3):
    wid = lax.axis_index("s") * NC + lax.axis_index("c")
    tiles_per_w = PER_W // (8 * SUB)
    sg = (sg0, sg1, sg2, sg3)
    so = (so0, so1, so2, so3)

    def fire_gather(c_dyn, itb, row, b):
        pltpu.async_copy(
            tbl_hbm.at[idx_v.at[itb, row]], rows_v.at[b], sg[b]
        )

    def fire_out(c_dyn, b):
        out_row = wid * PER_W + c_dyn * CHUNK
        pltpu.async_copy(rows_v.at[b], out_hbm.at[pl.ds(out_row, CHUNK)], so[b])

    def wait_gather(b):
        pltpu.make_async_copy(
            out_hbm.at[pl.ds(0, CHUNK)], rows_v.at[b], sg[b]
        ).wait()

    def wait_out(b):
        pltpu.make_async_copy(
            rows_v.at[b], out_hbm.at[pl.ds(0, CHUNK)], so[b]
        ).wait()

    pltpu.sync_copy(idx_hbm.at[wid * tiles_per_w], idx_v.at[0])
    fire_gather(0, 0, 0, 0)

    def body(k, carry):
        c0 = k * UNROLL
        for j in range(UNROLL):
            c = c0 + j
            b = j % NBUF
            itb = (j // 8) % 2
            if j % 8 == 0:
                pltpu.sync_copy(
                    idx_hbm.at[wid * tiles_per_w + c // 8], idx_v.at[itb]
                )
            @pl.when(c >= NBUF)
            def _(b=b):
                wait_out(b)

            if j == 0:
                @pl.when(k > 0)
                def _(itb=itb, b=b):
                    fire_gather(c, itb, j % 8, b)
            else:
                fire_gather(c, itb, j % 8, b)

            bp = (j - 1) % NBUF

            @pl.when(c > 0)
            def _(c=c, bp=bp):
                wait_gather(bp)
                fire_out(c - 1, bp)

        return carry

    lax.fori_loop(0, STEPS // UNROLL, body, 0)
    last_b = (STEPS - 1) % NBUF
    wait_gather(last_b)
    fire_out(STEPS - 1, last_b)
    for b in range(NBUF):
        wait_out(b)


def kernel(indices, glove_table, weight):
    combined = _combine(glove_table, weight)
    idx3d = indices.reshape(IDX_T, 8, SUB).astype(jnp.int32)
    out = pl.kernel(
        _gather_body,
        out_type=jax.ShapeDtypeStruct((N, DP), jnp.float32),
        mesh=plsc.VectorSubcoreMesh(core_axis_name="c", subcore_axis_name="s"),
        compiler_params=pltpu.CompilerParams(use_tc_tiling_on_sc=True),
        scratch_types=[
            pltpu.VMEM((2, 8, SUB), jnp.int32),
            pltpu.VMEM((NBUF, CHUNK, DP), jnp.float32),
        ] + [pltpu.SemaphoreType.DMA] * 8,
    )(idx3d, combined)
    return out[:, :D].reshape(B, L, D)
